# baseline (speedup 1.0000x reference)
import functools

import jax
import jax.numpy as jnp
from jax import lax
from jax.experimental import pallas as pl
from jax.experimental.pallas import tpu as pltpu
from jax.experimental.pallas import tpu_sc as plsc

S, H, W, D = 2048, 12, 32, 64
CS = 256
HALO = 128
NOB = 4

_info = plsc.get_sparse_core_info()
NC, NS, NL = _info.num_cores, _info.num_subcores, _info.num_lanes
NW = NC * NS
NSC = S // CS
NITEM = H * NSC
PER_W = NITEM // NW


def _body(src_hbm, dst_hbm, stage, obuf, sem_s):
    wid = lax.axis_index("s") * NC + lax.axis_index("c")
    NJ = CS // NL

    if True:
        def slot_body(slot, carry):
            m = wid * PER_W + slot
            h = m // NSC
            sc = lax.rem(m, NSC)
            s0 = sc * CS

            @pl.when(sc == 0)
            def _():
                pltpu.sync_copy(src_hbm.at[0, h, :, pl.ds(0, CS)],
                                stage.at[:, pl.ds(HALO, CS)])

            @pl.when(sc > 0)
            def _():
                pltpu.sync_copy(
                    src_hbm.at[0, h, :, pl.ds(s0 - HALO, CS + HALO)], stage)

            iota = lax.iota(jnp.int32, NL)
            for w in range(W):
                delta = (W - 1) - w
                r = w % NOB
                n = slot * W + w

                @pl.when(n >= NOB)
                def _():
                    pltpu.make_async_copy(obuf.at[0],
                                          dst_hbm.at[0, 0, :, pl.ds(0, CS)],
                                          sem_s).wait()

                def dbody(d, carry2):
                    for dd in range(2):
                        di = 2 * d + dd
                        for j in range(NJ):
                            obuf[r, di, pl.ds(NL * j, NL)] = (
                                stage[di, pl.ds(HALO + NL * j - delta, NL)])
                    return carry2

                lax.fori_loop(0, D // 2, dbody, 0)

                @pl.when(s0 == 0)
                def _():
                    masks = [(iota + NL * j < w, iota + NL * j < (W - 1))
                             for j in range(2)]

                    def fbody(d, carry2):
                        cval = stage[d, pl.ds(HALO + w, NL)][0]
                        for j in range(2):
                            m0, m1 = masks[j]
                            x = obuf[r, d, pl.ds(NL * j, NL)]
                            obuf[r, d, pl.ds(NL * j, NL)] = (
                                jnp.where(m0, 0.0, jnp.where(m1, cval, x)))
                        return carry2

                    lax.fori_loop(0, D, fbody, 0)

                pltpu.async_copy(obuf.at[r],
                                 dst_hbm.at[h, w, :, pl.ds(s0, CS)], sem_s)
            return carry

        lax.fori_loop(0, PER_W, slot_body, 0)

    for _ in range(NOB):
        pltpu.make_async_copy(obuf.at[0], dst_hbm.at[0, 0, :, pl.ds(0, CS)],
                              sem_s).wait()


@jax.jit
def _unfold(kT, vT):
    fn = functools.partial(
        pl.kernel,
        out_type=jax.ShapeDtypeStruct((H, W, D, S), jnp.float32),
        mesh=plsc.VectorSubcoreMesh(core_axis_name="c", subcore_axis_name="s"),
        scratch_types=[
            pltpu.VMEM((D, CS + HALO), jnp.float32),
            pltpu.VMEM((NOB, D, CS), jnp.float32),
            pltpu.SemaphoreType.DMA,
        ],
    )(_body)
    return fn(kT), fn(vT)


def kernel(k, v):
    kT = jnp.transpose(k, (0, 2, 3, 1))
    vT = jnp.transpose(v, (0, 2, 3, 1))
    kwT, vwT = _unfold(kT, vT)
    kw = jnp.transpose(kwT, (3, 0, 1, 2))[None]
    vw = jnp.transpose(vwT, (3, 0, 1, 2))[None]
    return kw, vw


# speedup vs baseline: 2.5686x; 2.5686x over previous
import functools

import jax
import jax.numpy as jnp
from jax import lax
from jax.experimental import pallas as pl
from jax.experimental.pallas import tpu as pltpu
from jax.experimental.pallas import tpu_sc as plsc

S, H, W, D = 2048, 12, 32, 64
PAD = W

_info = plsc.get_sparse_core_info()
NC, NS, NL = _info.num_cores, _info.num_subcores, _info.num_lanes
NW = NC * NS
CT = 32
NCHUNK = S // CT


def _sc_body(src_hbm, z_hbm, dst_hbm, stage, sem_g, sem_s):
    wid = lax.axis_index("s") * NC + lax.axis_index("c")

    def run_chunk(c, edge_chunk):
        t0 = c * CT

        if edge_chunk:
            @pl.when(c == 0)
            def _():
                for h in range(H):
                    pltpu.async_copy(src_hbm.at[0, pl.ds(0, CT), h, :],
                                     stage.at[h, pl.ds(PAD, CT), :], sem_g)
                for h in range(H):
                    pltpu.make_async_copy(src_hbm.at[0, pl.ds(0, CT), h, :],
                                          stage.at[h, pl.ds(PAD, CT), :],
                                          sem_g).wait()

            @pl.when(c > 0)
            def _():
                for h in range(H):
                    pltpu.async_copy(
                        src_hbm.at[0, pl.ds(t0 - PAD, CT + PAD), h, :],
                        stage.at[h], sem_g)
                for h in range(H):
                    pltpu.make_async_copy(
                        src_hbm.at[0, pl.ds(t0 - PAD, CT + PAD), h, :],
                        stage.at[h], sem_g).wait()
        else:
            for h in range(H):
                pltpu.async_copy(src_hbm.at[0, pl.ds(t0 - PAD, CT + PAD), h, :],
                                 stage.at[h], sem_g)
            for h in range(H):
                pltpu.make_async_copy(
                    src_hbm.at[0, pl.ds(t0 - PAD, CT + PAD), h, :],
                    stage.at[h], sem_g).wait()

        def scat(i, carry):
            t = t0 + i

            @pl.when(t >= W - 1)
            def _():
                pltpu.async_copy(stage.at[:, pl.ds(i + 1, W), :],
                                 dst_hbm.at[t], sem_s)

            return carry

        def drain(i, carry):
            t = t0 + i

            @pl.when(t >= W - 1)
            def _():
                pltpu.make_async_copy(stage.at[:, pl.ds(i + 1, W), :],
                                      dst_hbm.at[t], sem_s).wait()

            return carry

        lax.fori_loop(0, CT, scat, 0)
        lax.fori_loop(0, CT, drain, 0)

    def edge_window():
        @pl.when(wid < W - 1)
        def _():
            for h in range(H):
                pltpu.async_copy(src_hbm.at[0, pl.ds(0, W), h, :],
                                 stage.at[h, pl.ds(0, W), :], sem_g)
            for h in range(H):
                pltpu.make_async_copy(src_hbm.at[0, pl.ds(0, W), h, :],
                                      stage.at[h, pl.ds(0, W), :], sem_g).wait()
            pltpu.sync_copy(z_hbm, stage.at[:, pl.ds(wid + 1, W), :])
            pltpu.sync_copy(stage.at[:, pl.ds(0, W), :], dst_hbm.at[wid])

    run_chunk(wid, True)
    edge_window()
    run_chunk(wid + NW, False)


def _tc_body(vT_ref, out_ref):
    w = pl.program_id(1)
    x = vT_ref[0, 0]
    delta = (W - 1) - w
    rolled = pltpu.roll(x, delta, axis=1)
    t = lax.broadcasted_iota(jnp.int32, (D, S), 1)
    ccol = rolled[:, W - 1:W]
    out_ref[0, 0] = jnp.where(t < w, 0.0,
                              jnp.where(t < W - 1, ccol, rolled))


@jax.jit
def _unfold(k, v, z):
    kw_std = functools.partial(
        pl.kernel,
        out_type=jax.ShapeDtypeStruct((S, H, W, D), jnp.float32),
        mesh=plsc.VectorSubcoreMesh(core_axis_name="c", subcore_axis_name="s"),
        scratch_types=[
            pltpu.VMEM((H, CT + PAD, D), jnp.float32),
            pltpu.SemaphoreType.DMA,
            pltpu.SemaphoreType.DMA,
        ],
    )(_sc_body)(k, z)

    vT = jnp.transpose(v, (0, 2, 3, 1))
    vwT = pl.pallas_call(
        _tc_body,
        grid=(H, W),
        in_specs=[pl.BlockSpec((1, 1, D, S), lambda h, w: (0, h, 0, 0))],
        out_specs=pl.BlockSpec((1, 1, D, S), lambda h, w: (h, w, 0, 0)),
        out_shape=jax.ShapeDtypeStruct((H, W, D, S), jnp.float32),
    )(vT)
    return kw_std, vwT


def kernel(k, v):
    kw_std, vwT = _unfold(k, v, jnp.zeros((H, PAD, D), jnp.float32))
    kw = kw_std[None]
    vw = jnp.transpose(vwT, (3, 0, 1, 2))[None]
    return kw, vw


# speedup vs baseline: 3.0940x; 1.2046x over previous
import functools

import jax
import jax.numpy as jnp
from jax import lax
from jax.experimental import pallas as pl
from jax.experimental.pallas import tpu as pltpu
from jax.experimental.pallas import tpu_sc as plsc

S, H, W, D = 2048, 12, 32, 64
PAD = W

_info = plsc.get_sparse_core_info()
NC, NS, NL = _info.num_cores, _info.num_subcores, _info.num_lanes
NW = NC * NS
CT = 32
NCHUNK = S // CT


def _sc_body(src_hbm, z_hbm, dst_hbm, stage, sem_g, sem_s):
    wid = lax.axis_index("s") * NC + lax.axis_index("c")

    def run_chunk(c, edge_chunk):
        t0 = c * CT

        if edge_chunk:
            @pl.when(c == 0)
            def _():
                for h in range(H):
                    pltpu.async_copy(src_hbm.at[0, pl.ds(0, CT), h, :],
                                     stage.at[h, pl.ds(PAD, CT), :], sem_g)
                for h in range(H):
                    pltpu.make_async_copy(src_hbm.at[0, pl.ds(0, CT), h, :],
                                          stage.at[h, pl.ds(PAD, CT), :],
                                          sem_g).wait()

            @pl.when(c > 0)
            def _():
                for h in range(H):
                    pltpu.async_copy(
                        src_hbm.at[0, pl.ds(t0 - PAD, CT + PAD), h, :],
                        stage.at[h], sem_g)
                for h in range(H):
                    pltpu.make_async_copy(
                        src_hbm.at[0, pl.ds(t0 - PAD, CT + PAD), h, :],
                        stage.at[h], sem_g).wait()
        else:
            for h in range(H):
                pltpu.async_copy(src_hbm.at[0, pl.ds(t0 - PAD, CT + PAD), h, :],
                                 stage.at[h], sem_g)
            for h in range(H):
                pltpu.make_async_copy(
                    src_hbm.at[0, pl.ds(t0 - PAD, CT + PAD), h, :],
                    stage.at[h], sem_g).wait()

        def scat(i, carry):
            t = t0 + i

            @pl.when(t >= W - 1)
            def _():
                pltpu.async_copy(stage.at[:, pl.ds(i + 1, W), :],
                                 dst_hbm.at[t], sem_s)

            return carry

        def drain(i, carry):
            t = t0 + i

            @pl.when(t >= W - 1)
            def _():
                pltpu.make_async_copy(stage.at[:, pl.ds(i + 1, W), :],
                                      dst_hbm.at[t], sem_s).wait()

            return carry

        lax.fori_loop(0, CT, scat, 0)
        lax.fori_loop(0, CT, drain, 0)

    def edge_window():
        @pl.when(wid < W - 1)
        def _():
            for h in range(H):
                pltpu.async_copy(src_hbm.at[0, pl.ds(0, W), h, :],
                                 stage.at[h, pl.ds(0, W), :], sem_g)
            for h in range(H):
                pltpu.make_async_copy(src_hbm.at[0, pl.ds(0, W), h, :],
                                      stage.at[h, pl.ds(0, W), :], sem_g).wait()
            pltpu.sync_copy(z_hbm, stage.at[:, pl.ds(wid + 1, W), :])
            pltpu.sync_copy(stage.at[:, pl.ds(0, W), :], dst_hbm.at[wid])

    run_chunk(wid, True)
    edge_window()
    run_chunk(wid + NW, False)


def _tc_window(vT_ref, out_ref):
    x = vT_ref[0, 0]
    t128 = lax.broadcasted_iota(jnp.int32, (D, 128), 1)
    zpre = jnp.zeros((D, W), jnp.float32)
    xpad = jnp.concatenate([zpre, x[:, :128]], axis=1)
    for w in range(W):
        delta = (W - 1) - w
        out_ref[0, w, :, 128:S] = x[:, 128 - delta:S - delta]
        pre_shift = xpad[:, W - delta:W - delta + 128]
        ccol = x[:, w:w + 1]
        out_ref[0, w, :, 0:128] = jnp.where(
            t128 < w, 0.0, jnp.where(t128 < W - 1, ccol, pre_shift))


@jax.jit
def _unfold(k, v, z):
    vT = jnp.transpose(v, (0, 2, 3, 1))
    kw_std = functools.partial(
        pl.kernel,
        out_type=jax.ShapeDtypeStruct((S, H, W, D), jnp.float32),
        mesh=plsc.VectorSubcoreMesh(core_axis_name="c", subcore_axis_name="s"),
        scratch_types=[
            pltpu.VMEM((H, CT + PAD, D), jnp.float32),
            pltpu.SemaphoreType.DMA,
            pltpu.SemaphoreType.DMA,
        ],
    )(_sc_body)(k, z)

    vwT = pl.pallas_call(
        _tc_window,
        grid=(H,),
        in_specs=[pl.BlockSpec((1, 1, D, S), lambda h: (0, h, 0, 0))],
        out_specs=pl.BlockSpec((1, W, D, S), lambda h: (h, 0, 0, 0)),
        out_shape=jax.ShapeDtypeStruct((H, W, D, S), jnp.float32),
    )(vT)
    return kw_std, vwT


def kernel(k, v):
    kw_std, vwT = _unfold(k, v, jnp.zeros((H, PAD, D), jnp.float32))
    kw = kw_std[None]
    vw = jnp.transpose(vwT, (3, 0, 1, 2))[None]
    return kw, vw
